# baseline (speedup 1.0000x reference)
import jax
import jax.numpy as jnp
from jax.experimental import pallas as pl


def _body(x_ref, td_ref, tw_ref, out_ref):
    T = td_ref.shape[0]
    W = tw_ref.shape[0]
    G, L = x_ref.shape[2], x_ref.shape[3]
    NL = L // 3
    N = G * NL

    X = x_ref[0, 0]

    iota_l = jax.lax.broadcasted_iota(jnp.int32, (L, NL), 0)
    iota_n = jax.lax.broadcasted_iota(jnp.int32, (L, NL), 1)
    s_day = (iota_l == 3 * iota_n + 1).astype(jnp.float32)
    s_week = (iota_l == 3 * iota_n + 2).astype(jnp.float32)
    day16 = jax.lax.dot_general(X, s_day, (((1,), (0,)), ((), ())),
                                preferred_element_type=jnp.float32)
    week16 = jax.lax.dot_general(X, s_week, (((1,), (0,)), ((), ())),
                                 preferred_element_type=jnp.float32)

    d16 = jnp.clip(day16 * T, 0, T - 1).astype(jnp.int32)
    w16 = jnp.clip(week16, 0, W - 1).astype(jnp.int32)
    d_row = jnp.concatenate([d16[g:g + 1, :] for g in range(G)], axis=1)
    w_row = jnp.concatenate([w16[g:g + 1, :] for g in range(G)], axis=1)

    iota_t = jax.lax.broadcasted_iota(jnp.int32, (T, N), 0)
    oh_d = (iota_t == d_row).astype(jnp.float32)
    iota_w = jax.lax.broadcasted_iota(jnp.int32, (W, N), 0)
    oh_w = (iota_w == w_row).astype(jnp.float32)

    acc = jax.lax.dot_general(td_ref[...], oh_d, (((0,), (0,)), ((), ())),
                              preferred_element_type=jnp.float32)
    acc = acc + jax.lax.dot_general(tw_ref[...], oh_w, (((0,), (0,)), ((), ())),
                                    preferred_element_type=jnp.float32)
    out_ref[0, :, :] = acc


def kernel(x, time_day, time_week):
    B, S, N, C = x.shape
    T, F = time_day.shape
    W = time_week.shape[0]
    G, L = 16, (N * C) // 16
    xf = x.reshape(B, S, G, L)

    out = pl.pallas_call(
        _body,
        grid=(B,),
        in_specs=[
            pl.BlockSpec((1, 1, G, L), lambda b: (b, S - 1, 0, 0)),
            pl.BlockSpec((T, F), lambda b: (0, 0)),
            pl.BlockSpec((W, F), lambda b: (0, 0)),
        ],
        out_specs=pl.BlockSpec((1, F, N), lambda b: (b, 0, 0)),
        out_shape=jax.ShapeDtypeStruct((B, F, N), jnp.float32),
    )(xf, time_day, time_week)
    return out[..., None]
